# baseline (speedup 1.0000x reference)
import functools

import jax
import jax.numpy as jnp
from jax import lax
from jax.experimental import pallas as pl
from jax.experimental.pallas import tpu as pltpu
from jax.experimental.pallas import tpu_sc as plsc

NCLS = 50
BATCH = 64
NPTS = 4096
BPB = 8
TC_GRID = BATCH // BPB

NWORK = 32
RPW = BATCH // NWORK
NBINS = 64
ENC = 8192
UNROLL = 4


def _argmax_body(logits_ref, pred_ref):
    best = logits_ref[0]
    idx = jnp.zeros((BPB, NPTS), jnp.int32)
    for c in range(1, NCLS):
        v = logits_ref[c]
        gt = v > best
        best = jnp.where(gt, v, best)
        idx = jnp.where(gt, c, idx)
    pred_ref[...] = idx


def _tc_argmax(logits_cmajor):
    return pl.pallas_call(
        _argmax_body,
        grid=(TC_GRID,),
        in_specs=[pl.BlockSpec((NCLS, BPB, NPTS), lambda i: (0, i, 0))],
        out_specs=pl.BlockSpec((BPB, NPTS), lambda i: (i, 0)),
        out_shape=jax.ShapeDtypeStruct((BATCH, NPTS), jnp.int32),
    )(logits_cmajor)


_SC_MESH = dict(
    mesh=plsc.VectorSubcoreMesh(core_axis_name="c", subcore_axis_name="s"),
    compiler_params=pltpu.CompilerParams(needs_layout_passes=False),
)


@functools.partial(
    pl.kernel,
    out_type=jax.ShapeDtypeStruct((NWORK, NBINS), jnp.float32),
    scratch_types=[
        pltpu.VMEM((RPW, NPTS), jnp.int32),
        pltpu.VMEM((16 * NBINS,), jnp.float32),
        pltpu.VMEM((NBINS,), jnp.float32),
    ],
    **_SC_MESH,
)
def _sc_lab_hist(lab_hbm, out_hbm, lv, lhist, hist):
    cid = lax.axis_index("c")
    sid = lax.axis_index("s")
    w = sid * 2 + cid
    pltpu.sync_copy(lab_hbm.at[pl.ds(w * RPW, RPW)], lv)

    zeros16 = jnp.zeros((16,), jnp.float32)
    for j in range(16 * NBINS // 16):
        lhist[pl.ds(j * 16, 16)] = zeros16

    ones = jnp.ones((16,), jnp.float32)
    lbase = lax.iota(jnp.int32, 16) * NBINS

    for r in range(RPW):
        def body(k, carry):
            for u in range(UNROLL):
                off = k * (16 * UNROLL) + u * 16
                l = lv[r, pl.ds(off, 16)]
                plsc.addupdate_scatter(lhist, [lbase + l], ones)
            return carry
        lax.fori_loop(0, NPTS // (16 * UNROLL), body, 0)

    for j in range(NBINS // 16):
        acc = zeros16
        for t in range(16):
            acc = acc + lhist[pl.ds(t * NBINS + j * 16, 16)]
        hist[pl.ds(j * 16, 16)] = acc

    pltpu.sync_copy(hist, out_hbm.at[w])


@functools.partial(
    pl.kernel,
    out_type=jax.ShapeDtypeStruct((NWORK, 2 * NBINS), jnp.float32),
    scratch_types=[
        pltpu.VMEM((RPW, NPTS), jnp.int32),
        pltpu.VMEM((RPW, NPTS), jnp.int32),
        pltpu.VMEM((16 * NBINS,), jnp.float32),
        pltpu.VMEM((2 * NBINS,), jnp.float32),
    ],
    **_SC_MESH,
)
def _sc_pred_hist(pred_hbm, lab_hbm, out_hbm, pv, lv, lhist, hist):
    cid = lax.axis_index("c")
    sid = lax.axis_index("s")
    w = sid * 2 + cid
    pltpu.sync_copy(pred_hbm.at[pl.ds(w * RPW, RPW)], pv)
    pltpu.sync_copy(lab_hbm.at[pl.ds(w * RPW, RPW)], lv)

    zeros16 = jnp.zeros((16,), jnp.float32)
    for j in range(16 * NBINS // 16):
        lhist[pl.ds(j * 16, 16)] = zeros16

    enc = jnp.float32(ENC + 1)
    one = jnp.float32(1.0)
    lbase = lax.iota(jnp.int32, 16) * NBINS

    for r in range(RPW):
        def body(k, carry):
            for u in range(UNROLL):
                off = k * (16 * UNROLL) + u * 16
                p = pv[r, pl.ds(off, 16)]
                l = lv[r, pl.ds(off, 16)]
                w16 = jnp.where(p == l, enc, one)
                plsc.addupdate_scatter(lhist, [lbase + p], w16)
            return carry
        lax.fori_loop(0, NPTS // (16 * UNROLL), body, 0)

    for j in range(NBINS // 16):
        acc_i = zeros16
        acc_c = zeros16
        for t in range(16):
            vi = lhist[pl.ds(t * NBINS + j * 16, 16)].astype(jnp.int32)
            acc_i = acc_i + (vi >> 13).astype(jnp.float32)
            acc_c = acc_c + (vi & (ENC - 1)).astype(jnp.float32)
        hist[pl.ds(j * 16, 16)] = acc_i
        hist[pl.ds(NBINS + j * 16, 16)] = acc_c

    pltpu.sync_copy(hist, out_hbm.at[w])


@jax.jit
def kernel(seg_logits, seg_labels):
    lab_part = _sc_lab_hist(seg_labels)
    pred = _tc_argmax(jnp.transpose(seg_logits, (1, 0, 2)))
    pred_part = _sc_pred_hist(pred, seg_labels)
    lab_cnt = jnp.sum(lab_part, axis=0)
    pc = jnp.sum(pred_part, axis=0)
    inter = pc[0:NCLS]
    union = pc[NBINS:NBINS + NCLS] + lab_cnt[0:NCLS] - inter
    return inter, union


# speedup vs baseline: 1.1843x; 1.1843x over previous
import functools

import jax
import jax.numpy as jnp
from jax import lax
from jax.experimental import pallas as pl
from jax.experimental.pallas import tpu as pltpu
from jax.experimental.pallas import tpu_sc as plsc

NCLS = 50
BATCH = 64
NPTS = 4096
BPB = 8
TC_GRID = BATCH // BPB

NWORK = 32
RPW = BATCH // NWORK
NBINS = 64
ENC = 8192
UNROLL = 4


def _argmax_body(logits_ref, pred_ref):
    best = logits_ref[0]
    idx = jnp.zeros((BPB, NPTS), jnp.int32)
    for c in range(1, NCLS):
        v = logits_ref[c]
        gt = v > best
        best = jnp.where(gt, v, best)
        idx = jnp.where(gt, c, idx)
    pred_ref[...] = idx


def _tc_argmax(logits_cmajor):
    return pl.pallas_call(
        _argmax_body,
        grid=(TC_GRID,),
        in_specs=[pl.BlockSpec((NCLS, BPB, NPTS), lambda i: (0, i, 0))],
        out_specs=pl.BlockSpec((BPB, NPTS), lambda i: (i, 0)),
        out_shape=jax.ShapeDtypeStruct((BATCH, NPTS), jnp.int32),
    )(logits_cmajor)


_SC_MESH = dict(
    mesh=plsc.VectorSubcoreMesh(core_axis_name="c", subcore_axis_name="s"),
    compiler_params=pltpu.CompilerParams(needs_layout_passes=False),
)


@functools.partial(
    pl.kernel,
    out_type=jax.ShapeDtypeStruct((NWORK, 3 * NBINS), jnp.float32),
    scratch_types=[
        pltpu.VMEM((RPW, NPTS), jnp.int32),
        pltpu.VMEM((RPW, NPTS), jnp.int32),
        pltpu.VMEM((2 * 16 * NBINS,), jnp.float32),
        pltpu.VMEM((3 * NBINS,), jnp.float32),
    ],
    **_SC_MESH,
)
def _sc_hist(pred_hbm, lab_hbm, out_hbm, pv, lv, lhist, hist):
    cid = lax.axis_index("c")
    sid = lax.axis_index("s")
    w = sid * 2 + cid
    pltpu.sync_copy(pred_hbm.at[pl.ds(w * RPW, RPW)], pv)
    pltpu.sync_copy(lab_hbm.at[pl.ds(w * RPW, RPW)], lv)

    zeros16 = jnp.zeros((16,), jnp.float32)
    for j in range(2 * 16 * NBINS // 16):
        lhist[pl.ds(j * 16, 16)] = zeros16

    enc = jnp.float32(ENC + 1)
    one = jnp.float32(1.0)
    ones = jnp.ones((16,), jnp.float32)
    lbase = lax.iota(jnp.int32, 16) * NBINS

    for r in range(RPW):
        def body(k, carry):
            for u in range(UNROLL):
                off = k * (16 * UNROLL) + u * 16
                p = pv[r, pl.ds(off, 16)]
                l = lv[r, pl.ds(off, 16)]
                w16 = jnp.where(p == l, enc, one)
                plsc.addupdate_scatter(lhist, [lbase + p], w16)
                plsc.addupdate_scatter(lhist, [lbase + (l + 16 * NBINS)], ones)
            return carry
        lax.fori_loop(0, NPTS // (16 * UNROLL), body, 0)

    for j in range(NBINS // 16):
        acc_i = zeros16
        acc_c = zeros16
        acc_l = zeros16
        for t in range(16):
            vi = lhist[pl.ds(t * NBINS + j * 16, 16)].astype(jnp.int32)
            acc_i = acc_i + (vi >> 13).astype(jnp.float32)
            acc_c = acc_c + (vi & (ENC - 1)).astype(jnp.float32)
            acc_l = acc_l + lhist[pl.ds(16 * NBINS + t * NBINS + j * 16, 16)]
        hist[pl.ds(j * 16, 16)] = acc_i
        hist[pl.ds(NBINS + j * 16, 16)] = acc_c
        hist[pl.ds(2 * NBINS + j * 16, 16)] = acc_l

    pltpu.sync_copy(hist, out_hbm.at[w])


@jax.jit
def kernel(seg_logits, seg_labels):
    pred = _tc_argmax(jnp.transpose(seg_logits, (1, 0, 2)))
    part = jnp.sum(_sc_hist(pred, seg_labels), axis=0)
    inter = part[0:NCLS]
    union = part[NBINS:NBINS + NCLS] + part[2 * NBINS:2 * NBINS + NCLS] - inter
    return inter, union
